# baseline (speedup 1.0000x reference)
import functools

import jax
import jax.numpy as jnp
from jax import lax
from jax.experimental import pallas as pl
from jax.experimental.pallas import tpu as pltpu
from jax.experimental.pallas import tpu_sc as plsc

NC = 2
NS = 16
NW = NC * NS
LANES = 16
CHUNK = 128
BS = 8192


def _proj_block(tbl_ref, w_ref, out_ref):
  out_ref[...] = jnp.dot(
      tbl_ref[...], w_ref[...], preferred_element_type=jnp.float32)[:, 0]


@jax.jit
def _tc_project(table, w_col):
  V, D = table.shape
  grid = pl.cdiv(V, BS)
  return pl.pallas_call(
      _proj_block,
      grid=(grid,),
      in_specs=[
          pl.BlockSpec((BS, D), lambda i: (i, 0)),
          pl.BlockSpec((D, 1), lambda i: (0, 0)),
      ],
      out_specs=pl.BlockSpec((BS,), lambda i: (i,)),
      out_shape=jax.ShapeDtypeStruct((V,), jnp.float32),
  )(table, w_col)


@jax.jit
def _sc_pool(x_t, p, b16):
  L, B = x_t.shape
  bpw = B // NW
  nchunk = bpw // CHUNK

  mesh = plsc.VectorSubcoreMesh(core_axis_name="c", subcore_axis_name="s")

  @functools.partial(
      pl.kernel,
      out_type=jax.ShapeDtypeStruct((B,), jnp.float32),
      mesh=mesh,
      compiler_params=pltpu.CompilerParams(use_tc_tiling_on_sc=False),
      scratch_types=[
          pltpu.VMEM((L, bpw), jnp.int32),
          pltpu.VMEM((bpw,), jnp.float32),
          pltpu.VMEM((bpw,), jnp.float32),
          pltpu.VMEM((bpw,), jnp.float32),
          pltpu.VMEM((LANES,), jnp.float32),
          pltpu.VMEM((bpw,), jnp.float32),
          pltpu.SemaphoreType.DMA,
          pltpu.SemaphoreType.DMA,
      ],
  )
  def k(x_hbm, p_hbm, b_hbm, out_hbm, x_v, g0, g1, acc, b_v, out_v, sem0,
        sem1):
    wid = lax.axis_index("s") * NC + lax.axis_index("c")
    base = wid * bpw
    pltpu.sync_copy(x_hbm.at[:, pl.ds(base, bpw)], x_v)
    pltpu.sync_copy(b_hbm, b_v)

    zero = jnp.zeros((LANES,), jnp.float32)
    for blk in range(bpw // LANES):
      acc[pl.ds(blk * LANES, LANES)] = zero

    def fire(l, g, sem):
      for c in range(nchunk):
        idx = x_v.at[l, pl.ds(c * CHUNK, CHUNK)]
        dst = g.at[pl.ds(c * CHUNK, CHUNK)]
        pltpu.async_copy(p_hbm.at[idx], dst, sem)

    def drain(g, sem):
      pltpu.make_async_copy(p_hbm.at[pl.ds(0, bpw)], g, sem).wait()

    def accumulate(g):
      for blk in range(bpw // LANES):
        o = blk * LANES
        acc[pl.ds(o, LANES)] = acc[pl.ds(o, LANES)] + g[pl.ds(o, LANES)]

    fire(0, g0, sem0)
    fire(1, g1, sem1)

    def step(pr, carry):
      l = 2 * pr
      drain(g0, sem0)
      accumulate(g0)

      @pl.when(l + 2 < L)
      def _():
        fire(l + 2, g0, sem0)

      drain(g1, sem1)
      accumulate(g1)

      @pl.when(l + 3 < L)
      def _():
        fire(l + 3, g1, sem1)

      return carry

    lax.fori_loop(0, (L + 1) // 2, step, 0)

    inv_l = jnp.float32(1.0 / L)
    bias_vec = b_v[pl.ds(0, LANES)]
    for blk in range(bpw // LANES):
      o = blk * LANES
      out_v[pl.ds(o, LANES)] = acc[pl.ds(o, LANES)] * inv_l + bias_vec

    pltpu.sync_copy(out_v, out_hbm.at[pl.ds(base, bpw)])

  return k(x_t, p, b16)


def kernel(x, table, W, b):
  x_t = jnp.transpose(x.astype(jnp.int32))
  w_col = W.reshape(1, -1).T.astype(jnp.float32)
  b16 = jnp.broadcast_to(b.reshape(-1)[:1], (LANES,)).astype(jnp.float32)
  p = _tc_project(table, w_col)
  return _sc_pool(x_t, p, b16)


# speedup vs baseline: 1.0156x; 1.0156x over previous
import functools

import jax
import jax.numpy as jnp
from jax import lax
from jax.experimental import pallas as pl
from jax.experimental.pallas import tpu as pltpu
from jax.experimental.pallas import tpu_sc as plsc

NC = 2
NS = 16
NW = NC * NS
LANES = 16
CHUNK = 128
BS = 32768


def _proj_block(tbl_ref, w_ref, out_ref):
  out_ref[...] = jnp.dot(
      tbl_ref[...], w_ref[...], preferred_element_type=jnp.float32)[:, 0]


@jax.jit
def _tc_project(table, w_col):
  V, D = table.shape
  grid = pl.cdiv(V, BS)
  return pl.pallas_call(
      _proj_block,
      grid=(grid,),
      in_specs=[
          pl.BlockSpec((BS, D), lambda i: (i, 0)),
          pl.BlockSpec((D, 1), lambda i: (0, 0)),
      ],
      out_specs=pl.BlockSpec((BS,), lambda i: (i,)),
      out_shape=jax.ShapeDtypeStruct((V,), jnp.float32),
  )(table, w_col)


@jax.jit
def _sc_pool(x_t, p, b16):
  L, B = x_t.shape
  bpw = B // NW
  nchunk = bpw // CHUNK

  mesh = plsc.VectorSubcoreMesh(core_axis_name="c", subcore_axis_name="s")

  @functools.partial(
      pl.kernel,
      out_type=jax.ShapeDtypeStruct((B,), jnp.float32),
      mesh=mesh,
      compiler_params=pltpu.CompilerParams(use_tc_tiling_on_sc=False),
      scratch_types=[
          pltpu.VMEM((L, bpw), jnp.int32),
          pltpu.VMEM((bpw,), jnp.float32),
          pltpu.VMEM((bpw,), jnp.float32),
          pltpu.VMEM((bpw,), jnp.float32),
          pltpu.VMEM((LANES,), jnp.float32),
          pltpu.VMEM((bpw,), jnp.float32),
          pltpu.SemaphoreType.DMA,
          pltpu.SemaphoreType.DMA,
      ],
  )
  def k(x_hbm, p_hbm, b_hbm, out_hbm, x_v, g0, g1, acc, b_v, out_v, sem0,
        sem1):
    wid = lax.axis_index("s") * NC + lax.axis_index("c")
    base = wid * bpw
    pltpu.sync_copy(x_hbm.at[:, pl.ds(base, bpw)], x_v)
    pltpu.sync_copy(b_hbm, b_v)

    zero = jnp.zeros((LANES,), jnp.float32)
    for blk in range(bpw // LANES):
      acc[pl.ds(blk * LANES, LANES)] = zero

    def fire(l, g, sem):
      for c in range(nchunk):
        idx = x_v.at[l, pl.ds(c * CHUNK, CHUNK)]
        dst = g.at[pl.ds(c * CHUNK, CHUNK)]
        pltpu.async_copy(p_hbm.at[idx], dst, sem)

    def drain(g, sem):
      pltpu.make_async_copy(p_hbm.at[pl.ds(0, bpw)], g, sem).wait()

    def accumulate(g):
      for blk in range(bpw // LANES):
        o = blk * LANES
        acc[pl.ds(o, LANES)] = acc[pl.ds(o, LANES)] + g[pl.ds(o, LANES)]

    fire(0, g0, sem0)
    fire(1, g1, sem1)

    def step(pr, carry):
      l = 2 * pr
      drain(g0, sem0)
      accumulate(g0)

      @pl.when(l + 2 < L)
      def _():
        fire(l + 2, g0, sem0)

      drain(g1, sem1)
      accumulate(g1)

      @pl.when(l + 3 < L)
      def _():
        fire(l + 3, g1, sem1)

      return carry

    lax.fori_loop(0, (L + 1) // 2, step, 0)

    inv_l = jnp.float32(1.0 / L)
    bias_vec = b_v[pl.ds(0, LANES)]
    for blk in range(bpw // LANES):
      o = blk * LANES
      out_v[pl.ds(o, LANES)] = acc[pl.ds(o, LANES)] * inv_l + bias_vec

    pltpu.sync_copy(out_v, out_hbm.at[pl.ds(base, bpw)])

  return k(x_t, p, b16)


def kernel(x, table, W, b):
  x_t = jnp.transpose(x.astype(jnp.int32))
  w_col = W.reshape(1, -1).T.astype(jnp.float32)
  b16 = jnp.broadcast_to(b.reshape(-1)[:1], (LANES,)).astype(jnp.float32)
  p = _tc_project(table, w_col)
  return _sc_pool(x_t, p, b16)


# speedup vs baseline: 1.3435x; 1.3229x over previous
import functools

import jax
import jax.numpy as jnp
from jax import lax
from jax.experimental import pallas as pl
from jax.experimental.pallas import tpu as pltpu
from jax.experimental.pallas import tpu_sc as plsc

NC = 2
NS = 16
NW = NC * NS
LANES = 16
CHUNK = 128


@jax.jit
def _sc_embed_pool_linear(x, table, w2, b16):
  B, L = x.shape
  V, D = table.shape
  bpw = B // NW
  nchunk = bpw // CHUNK
  nblk = bpw // LANES

  mesh = plsc.VectorSubcoreMesh(core_axis_name="c", subcore_axis_name="s")

  @functools.partial(
      pl.kernel,
      out_type=jax.ShapeDtypeStruct((B,), jnp.float32),
      mesh=mesh,
      compiler_params=pltpu.CompilerParams(
          needs_layout_passes=False, use_tc_tiling_on_sc=False),
      scratch_types=[
          pltpu.VMEM((bpw, L), jnp.int32),
          pltpu.VMEM((L, bpw), jnp.int32),
          pltpu.VMEM((bpw, D), jnp.float32),
          pltpu.VMEM((bpw, D), jnp.float32),
          pltpu.VMEM((2 * D,), jnp.float32),
          pltpu.VMEM((LANES,), jnp.float32),
          pltpu.VMEM((bpw,), jnp.float32),
          pltpu.SemaphoreType.DMA,
          pltpu.SemaphoreType.DMA,
      ],
  )
  def k(x_hbm, table_hbm, w_hbm, b_hbm, out_hbm, xr_v, xt_v, acc0, acc1,
        w_v, b_v, out_v, sem0, sem1):
    wid = lax.axis_index("s") * NC + lax.axis_index("c")
    base = wid * bpw
    pltpu.sync_copy(x_hbm.at[pl.ds(base, bpw), :], xr_v)
    pltpu.sync_copy(w_hbm, w_v)
    pltpu.sync_copy(b_hbm, b_v)

    lanes = lax.iota(jnp.int32, LANES)

    def tr(l, carry):
      col = jnp.broadcast_to(l, (LANES,))
      for blk in range(nblk):
        row = blk * LANES + lanes
        xt_v[l, pl.ds(blk * LANES, LANES)] = plsc.load_gather(
            xr_v, [row, col])
      return carry

    lax.fori_loop(0, L, tr, 0)

    def fire(l, acc, sem, add):
      for c in range(nchunk):
        idx = xt_v.at[l, pl.ds(c * CHUNK, CHUNK)]
        dst = acc.at[pl.ds(c * CHUNK, CHUNK), :]
        pltpu.async_copy(table_hbm.at[idx], dst, sem, add=add)

    def drain(acc, sem):
      pltpu.make_async_copy(table_hbm.at[pl.ds(0, bpw), :], acc, sem).wait()

    fire(0, acc0, sem0, add=False)
    fire(1, acc1, sem1, add=False)

    def step(pr, carry):
      l = 2 * pr
      drain(acc0, sem0)
      fire(l, acc0, sem0, add=True)
      drain(acc1, sem1)
      fire(l + 1, acc1, sem1, add=True)
      return carry

    lax.fori_loop(1, L // 2, step, 0)
    if L % 2:
      drain(acc0, sem0)
      fire(L - 1, acc0, sem0, add=True)
    drain(acc0, sem0)
    drain(acc1, sem1)

    inv_l = jnp.float32(1.0 / L)
    bias_vec = b_v[pl.ds(0, LANES)]

    def fin(blk, carry):
      c0 = blk * LANES
      row = c0 + lanes
      accv = jnp.zeros((LANES,), jnp.float32)
      for d0 in range(D):
        col = lax.rem(d0 + lanes, D)
        g = plsc.load_gather(acc0, [row, col]) + plsc.load_gather(
            acc1, [row, col])
        accv = accv + g * w_v[pl.ds(d0, LANES)]
      out_v[pl.ds(c0, LANES)] = accv * inv_l + bias_vec
      return carry

    lax.fori_loop(0, nblk, fin, 0)
    pltpu.sync_copy(out_v, out_hbm.at[pl.ds(base, bpw)])

  return k(x, table, w2, b16)


def kernel(x, table, W, b):
  w = W.reshape(-1).astype(jnp.float32)
  w2 = jnp.concatenate([w, w])
  b16 = jnp.broadcast_to(b.reshape(-1)[:1], (LANES,)).astype(jnp.float32)
  return _sc_embed_pool_linear(x.astype(jnp.int32), table, w2, b16)
